# initial kernel scaffold (unmeasured)
import jax
import jax.numpy as jnp
from jax import lax
from jax.experimental import pallas as pl
from jax.experimental.pallas import tpu as pltpu

N_DEV = 4

_CompilerParams = getattr(pltpu, "CompilerParams", None) or getattr(
    pltpu, "TPUCompilerParams"
)


def kernel(x, W1, W2):
    m, _ = x.shape
    d = W1.shape[1]
    n = W2.shape[1]
    chunk = m // N_DEV

    def body(
        x_ref, w1_ref, w2_ref, out_ref,
        h_ref, rs_send, rs_recv, ag_own, ag_recv,
        rs_send_sems, rs_recv_sems, ag_send_sems, ag_recv_sems,
    ):
        my = lax.axis_index("i")
        left = (my - 1) % N_DEV
        right = (my + 1) % N_DEV

        barrier_sem = pltpu.get_barrier_semaphore()
        for nbr in (left, right):
            pl.semaphore_signal(
                barrier_sem, inc=1,
                device_id=(nbr,), device_id_type=pl.DeviceIdType.MESH,
            )
        pl.semaphore_wait(barrier_sem, 2)

        xb = x_ref[...].astype(jnp.bfloat16)
        w1b = w1_ref[...].astype(jnp.bfloat16)
        h_ref[...] = jnp.dot(
            xb, w1b, preferred_element_type=jnp.float32
        ).astype(jnp.bfloat16)

        for s in range(N_DEV - 1):
            send_idx = (my - 1 - s) % N_DEV
            if s == 0:
                src = h_ref.at[pl.ds(send_idx * chunk, chunk), :]
            else:
                src = rs_send.at[s - 1]
            rdma = pltpu.make_async_remote_copy(
                src_ref=src,
                dst_ref=rs_recv.at[s],
                send_sem=rs_send_sems.at[s],
                recv_sem=rs_recv_sems.at[s],
                device_id=(right,),
                device_id_type=pl.DeviceIdType.MESH,
            )
            rdma.start()
            rdma.wait()
            recv_idx = (my - 2 - s) % N_DEV
            acc = rs_recv[s] + h_ref[pl.ds(recv_idx * chunk, chunk), :]
            if s < N_DEV - 2:
                rs_send[s] = acc
            else:
                ag_own[...] = acc

        w2b = w2_ref[...].astype(jnp.bfloat16)
        for t in range(N_DEV - 1):
            src = ag_own if t == 0 else ag_recv.at[t - 1]
            rdma = pltpu.make_async_remote_copy(
                src_ref=src,
                dst_ref=ag_recv.at[t],
                send_sem=ag_send_sems.at[t],
                recv_sem=ag_recv_sems.at[t],
                device_id=(right,),
                device_id_type=pl.DeviceIdType.MESH,
            )
            rdma.start()
            hold_idx = (my - t) % N_DEV
            held = ag_own[...] if t == 0 else ag_recv[t - 1]
            out_ref[pl.ds(hold_idx * chunk, chunk), :] = jnp.dot(
                held, w2b, preferred_element_type=jnp.float32
            )
            rdma.wait()
        last_idx = (my - (N_DEV - 1)) % N_DEV
        out_ref[pl.ds(last_idx * chunk, chunk), :] = jnp.dot(
            ag_recv[N_DEV - 2], w2b, preferred_element_type=jnp.float32
        )

    return pl.pallas_call(
        body,
        out_shape=jax.ShapeDtypeStruct((m, n), jnp.float32),
        in_specs=[
            pl.BlockSpec(memory_space=pltpu.VMEM),
            pl.BlockSpec(memory_space=pltpu.VMEM),
            pl.BlockSpec(memory_space=pltpu.VMEM),
        ],
        out_specs=pl.BlockSpec(memory_space=pltpu.VMEM),
        scratch_shapes=[
            pltpu.VMEM((m, d), jnp.bfloat16),
            pltpu.VMEM((N_DEV - 2, chunk, d), jnp.bfloat16),
            pltpu.VMEM((N_DEV - 1, chunk, d), jnp.bfloat16),
            pltpu.VMEM((chunk, d), jnp.bfloat16),
            pltpu.VMEM((N_DEV - 1, chunk, d), jnp.bfloat16),
            pltpu.SemaphoreType.DMA((N_DEV - 1,)),
            pltpu.SemaphoreType.DMA((N_DEV - 1,)),
            pltpu.SemaphoreType.DMA((N_DEV - 1,)),
            pltpu.SemaphoreType.DMA((N_DEV - 1,)),
        ],
        compiler_params=_CompilerParams(collective_id=0),
    )(x, W1, W2)


# baseline (device time: 181253 ns/iter reference)
import jax
import jax.numpy as jnp
from jax import lax
from jax.experimental import pallas as pl
from jax.experimental.pallas import tpu as pltpu

N_DEV = 4

_CompilerParams = getattr(pltpu, "CompilerParams", None) or getattr(
    pltpu, "TPUCompilerParams"
)


def kernel(x, W1, W2):
    m, _ = x.shape
    d = W1.shape[1]
    n = W2.shape[1]
    chunk = m // N_DEV

    xb = x.astype(jnp.bfloat16)
    w1b = W1.astype(jnp.bfloat16)
    w2b = W2.astype(jnp.bfloat16)

    def body(
        x_ref, w1_ref, w2_ref, out_ref,
        h_ref, rs_recv, ag_recv,
        rs_send_sems, rs_recv_sems, ag_send_sems, ag_recv_sems,
    ):
        my = lax.axis_index("i")
        left = (my - 1) % N_DEV
        right = (my + 1) % N_DEV

        barrier_sem = pltpu.get_barrier_semaphore()
        for nbr in (left, right):
            pl.semaphore_signal(
                barrier_sem, inc=1,
                device_id=(nbr,), device_id_type=pl.DeviceIdType.MESH,
            )
        pl.semaphore_wait(barrier_sem, 2)

        def h_chunk(idx):
            return h_ref.at[pl.ds(idx * chunk, chunk), :]

        def gemm1_chunk(idx):
            h_ref[pl.ds(idx * chunk, chunk), :] = jnp.dot(
                x_ref[pl.ds(idx * chunk, chunk), :], w1_ref[...],
                preferred_element_type=jnp.float32,
            ).astype(jnp.bfloat16)

        gemm1_chunk((my - 1) % N_DEV)

        rs = []
        for s in range(N_DEV - 1):
            send_idx = (my - 1 - s) % N_DEV
            rdma = pltpu.make_async_remote_copy(
                src_ref=h_chunk(send_idx),
                dst_ref=rs_recv.at[s],
                send_sem=rs_send_sems.at[s],
                recv_sem=rs_recv_sems.at[s],
                device_id=(right,),
                device_id_type=pl.DeviceIdType.MESH,
            )
            rdma.start()
            rs.append(rdma)
            if s == 0:
                for o in (2, 3, 4):
                    gemm1_chunk((my - o) % N_DEV)
            rdma.wait()
            recv_idx = (my - 2 - s) % N_DEV
            h_ref[pl.ds(recv_idx * chunk, chunk), :] = (
                rs_recv[s] + h_ref[pl.ds(recv_idx * chunk, chunk), :]
            )

        for t in range(N_DEV - 1):
            src = h_chunk(my) if t == 0 else ag_recv.at[t - 1]
            rdma = pltpu.make_async_remote_copy(
                src_ref=src,
                dst_ref=ag_recv.at[t],
                send_sem=ag_send_sems.at[t],
                recv_sem=ag_recv_sems.at[t],
                device_id=(right,),
                device_id_type=pl.DeviceIdType.MESH,
            )
            rdma.start()
            hold_idx = (my - t) % N_DEV
            held = h_ref[pl.ds(my * chunk, chunk), :] if t == 0 else ag_recv[t - 1]
            out_ref[pl.ds(hold_idx * chunk, chunk), :] = jnp.dot(
                held, w2_ref[...], preferred_element_type=jnp.float32
            ).astype(jnp.bfloat16)
            rdma.wait()
        last_idx = (my - (N_DEV - 1)) % N_DEV
        out_ref[pl.ds(last_idx * chunk, chunk), :] = jnp.dot(
            ag_recv[N_DEV - 2], w2_ref[...], preferred_element_type=jnp.float32
        ).astype(jnp.bfloat16)

    return pl.pallas_call(
        body,
        out_shape=jax.ShapeDtypeStruct((m, n), jnp.bfloat16),
        in_specs=[
            pl.BlockSpec(memory_space=pltpu.VMEM),
            pl.BlockSpec(memory_space=pltpu.VMEM),
            pl.BlockSpec(memory_space=pltpu.VMEM),
        ],
        out_specs=pl.BlockSpec(memory_space=pltpu.VMEM),
        scratch_shapes=[
            pltpu.VMEM((m, d), jnp.bfloat16),
            pltpu.VMEM((N_DEV - 1, chunk, d), jnp.bfloat16),
            pltpu.VMEM((N_DEV - 1, chunk, d), jnp.bfloat16),
            pltpu.SemaphoreType.DMA((N_DEV - 1,)),
            pltpu.SemaphoreType.DMA((N_DEV - 1,)),
            pltpu.SemaphoreType.DMA((N_DEV - 1,)),
            pltpu.SemaphoreType.DMA((N_DEV - 1,)),
        ],
        compiler_params=_CompilerParams(
            collective_id=0, vmem_limit_bytes=42 * 1024 * 1024
        ),
    )(xb, w1b, w2b)


# device time: 116222 ns/iter; 1.5595x vs baseline; 1.5595x over previous
import jax
import jax.numpy as jnp
from jax import lax
from jax.experimental import pallas as pl
from jax.experimental.pallas import tpu as pltpu

N_DEV = 4

_CompilerParams = getattr(pltpu, "CompilerParams", None) or getattr(
    pltpu, "TPUCompilerParams"
)


def kernel(x, W1, W2):
    m, _ = x.shape
    d = W1.shape[1]
    n = W2.shape[1]
    chunk = m // N_DEV
    half = chunk // 2

    xb = x.astype(jnp.bfloat16)
    w1b = W1.astype(jnp.bfloat16)
    w2b = W2.astype(jnp.bfloat16)

    def body(
        x_ref, w1_ref, w2_ref, out_ref,
        h_ref, rs_recv_a, rs_recv_b, ag_recv_a, ag_recv_b,
        rs_send_sems_a, rs_recv_sems_a, rs_send_sems_b, rs_recv_sems_b,
        ag_send_sems_a, ag_recv_sems_a, ag_send_sems_b, ag_recv_sems_b,
    ):
        my = lax.axis_index("i")
        left = (my - 1) % N_DEV
        right = (my + 1) % N_DEV

        barrier_sem = pltpu.get_barrier_semaphore()
        for nbr in (left, right):
            pl.semaphore_signal(
                barrier_sem, inc=1,
                device_id=(nbr,), device_id_type=pl.DeviceIdType.MESH,
            )
        pl.semaphore_wait(barrier_sem, 2)

        def h_a(idx):
            return h_ref.at[pl.ds(idx * chunk, half), :]

        def h_b(idx):
            return h_ref.at[pl.ds(idx * chunk + half, half), :]

        def gemm1_chunk(idx):
            h_ref[pl.ds(idx * chunk, chunk), :] = jnp.dot(
                x_ref[pl.ds(idx * chunk, chunk), :], w1_ref[...],
                preferred_element_type=jnp.float32,
            ).astype(jnp.bfloat16)

        def copy(dir_right, src, dst, send_sem, recv_sem):
            return pltpu.make_async_remote_copy(
                src_ref=src, dst_ref=dst,
                send_sem=send_sem, recv_sem=recv_sem,
                device_id=(right if dir_right else left,),
                device_id_type=pl.DeviceIdType.MESH,
            )

        gemm1_chunk((my - 1) % N_DEV)
        gemm1_chunk((my + 1) % N_DEV)

        for s in range(N_DEV - 1):
            ra = copy(
                True, h_a((my - 1 - s) % N_DEV), rs_recv_a.at[s],
                rs_send_sems_a.at[s], rs_recv_sems_a.at[s],
            )
            rb = copy(
                False, h_b((my + 1 + s) % N_DEV), rs_recv_b.at[s],
                rs_send_sems_b.at[s], rs_recv_sems_b.at[s],
            )
            ra.start()
            rb.start()
            if s == 0:
                gemm1_chunk((my + 2) % N_DEV)
                gemm1_chunk(my)
            recv_a = (my - 2 - s) % N_DEV
            recv_b = (my + 2 + s) % N_DEV
            ra.wait()
            h_ref[pl.ds(recv_a * chunk, half), :] = (
                rs_recv_a[s] + h_ref[pl.ds(recv_a * chunk, half), :]
            )
            rb.wait()
            h_ref[pl.ds(recv_b * chunk + half, half), :] = (
                rs_recv_b[s] + h_ref[pl.ds(recv_b * chunk + half, half), :]
            )

        def gemm2_store(row_start, held):
            out_ref[pl.ds(row_start, half), :] = jnp.dot(
                held, w2_ref[...], preferred_element_type=jnp.float32
            ).astype(jnp.bfloat16)

        for t in range(N_DEV - 1):
            ga = copy(
                True, h_a(my) if t == 0 else ag_recv_a.at[t - 1],
                ag_recv_a.at[t], ag_send_sems_a.at[t], ag_recv_sems_a.at[t],
            )
            gb = copy(
                False, h_b(my) if t == 0 else ag_recv_b.at[t - 1],
                ag_recv_b.at[t], ag_send_sems_b.at[t], ag_recv_sems_b.at[t],
            )
            ga.start()
            gb.start()
            hold_a = (my - t) % N_DEV
            hold_b = (my + t) % N_DEV
            held_a = (
                h_ref[pl.ds(my * chunk, half), :] if t == 0 else ag_recv_a[t - 1]
            )
            held_b = (
                h_ref[pl.ds(my * chunk + half, half), :]
                if t == 0
                else ag_recv_b[t - 1]
            )
            gemm2_store(hold_a * chunk, held_a)
            gemm2_store(hold_b * chunk + half, held_b)
            ga.wait()
            gb.wait()
        last = N_DEV - 1
        gemm2_store(((my - last) % N_DEV) * chunk, ag_recv_a[last - 1])
        gemm2_store(((my + last) % N_DEV) * chunk + half, ag_recv_b[last - 1])

    return pl.pallas_call(
        body,
        out_shape=jax.ShapeDtypeStruct((m, n), jnp.bfloat16),
        in_specs=[
            pl.BlockSpec(memory_space=pltpu.VMEM),
            pl.BlockSpec(memory_space=pltpu.VMEM),
            pl.BlockSpec(memory_space=pltpu.VMEM),
        ],
        out_specs=pl.BlockSpec(memory_space=pltpu.VMEM),
        scratch_shapes=[
            pltpu.VMEM((m, d), jnp.bfloat16),
            pltpu.VMEM((N_DEV - 1, half, d), jnp.bfloat16),
            pltpu.VMEM((N_DEV - 1, half, d), jnp.bfloat16),
            pltpu.VMEM((N_DEV - 1, half, d), jnp.bfloat16),
            pltpu.VMEM((N_DEV - 1, half, d), jnp.bfloat16),
            pltpu.SemaphoreType.DMA((N_DEV - 1,)),
            pltpu.SemaphoreType.DMA((N_DEV - 1,)),
            pltpu.SemaphoreType.DMA((N_DEV - 1,)),
            pltpu.SemaphoreType.DMA((N_DEV - 1,)),
            pltpu.SemaphoreType.DMA((N_DEV - 1,)),
            pltpu.SemaphoreType.DMA((N_DEV - 1,)),
            pltpu.SemaphoreType.DMA((N_DEV - 1,)),
            pltpu.SemaphoreType.DMA((N_DEV - 1,)),
        ],
        compiler_params=_CompilerParams(
            collective_id=0, vmem_limit_bytes=42 * 1024 * 1024
        ),
    )(xb, w1b, w2b)


# device time: 105682 ns/iter; 1.7151x vs baseline; 1.0997x over previous
import jax
import jax.numpy as jnp
from jax import lax
from jax.experimental import pallas as pl
from jax.experimental.pallas import tpu as pltpu

N_DEV = 4
N_Q = 4

_CompilerParams = getattr(pltpu, "CompilerParams", None) or getattr(
    pltpu, "TPUCompilerParams"
)


def kernel(x, W1, W2):
    m, _ = x.shape
    d = W1.shape[1]
    n = W2.shape[1]
    chunk = m // N_DEV
    q_rows = chunk // N_Q

    xb = x.astype(jnp.bfloat16)
    w1b = W1.astype(jnp.bfloat16)
    w2b = W2.astype(jnp.bfloat16)

    def body(
        x_ref, w1_ref, w2_ref, out_ref,
        h_ref, rs_recv, ag_recv,
        rs_send_sems, rs_recv_sems, ag_send_sems, ag_recv_sems,
    ):
        my = lax.axis_index("i")
        left = (my - 1) % N_DEV
        right = (my + 1) % N_DEV

        barrier_sem = pltpu.get_barrier_semaphore()
        for nbr in (left, right):
            pl.semaphore_signal(
                barrier_sem, inc=1,
                device_id=(nbr,), device_id_type=pl.DeviceIdType.MESH,
            )
        pl.semaphore_wait(barrier_sem, 2)

        def row0(c, q):
            return c * chunk + q * q_rows

        def h_q(c, q):
            return h_ref.at[pl.ds(row0(c, q), q_rows), :]

        def gemm1_chunk(idx):
            h_ref[pl.ds(idx * chunk, chunk), :] = jnp.dot(
                x_ref[pl.ds(idx * chunk, chunk), :], w1_ref[...],
                preferred_element_type=jnp.float32,
            ).astype(jnp.bfloat16)

        def rs_send_idx(q, s):
            return (my - 1 - s) % N_DEV if q < 2 else (my + 1 + s) % N_DEV

        def rs_recv_idx(q, s):
            return (my - 2 - s) % N_DEV if q < 2 else (my + 2 + s) % N_DEV

        def ag_hold_idx(q, t):
            return (my - t) % N_DEV if q < 2 else (my + t) % N_DEV

        def nbr_of(q):
            return right if q < 2 else left

        def start_rs(q, s):
            rdma = pltpu.make_async_remote_copy(
                src_ref=h_q(rs_send_idx(q, s), q),
                dst_ref=rs_recv.at[q, s],
                send_sem=rs_send_sems.at[q, s],
                recv_sem=rs_recv_sems.at[q, s],
                device_id=(nbr_of(q),),
                device_id_type=pl.DeviceIdType.MESH,
            )
            rdma.start()
            return rdma

        def start_ag(q, t):
            rdma = pltpu.make_async_remote_copy(
                src_ref=h_q(my, q) if t == 0 else ag_recv.at[q, t - 1],
                dst_ref=ag_recv.at[q, t],
                send_sem=ag_send_sems.at[q, t],
                recv_sem=ag_recv_sems.at[q, t],
                device_id=(nbr_of(q),),
                device_id_type=pl.DeviceIdType.MESH,
            )
            rdma.start()
            return rdma

        def gemm2_store(out_row, held):
            out_ref[pl.ds(out_row, q_rows), :] = jnp.dot(
                held, w2_ref[...], preferred_element_type=jnp.float32
            ).astype(jnp.bfloat16)

        CHAINS = (0, 2, 1, 3)

        gemm1_chunk((my - 1) % N_DEV)
        gemm1_chunk((my + 1) % N_DEV)
        rs = {}
        for q in CHAINS:
            rs[q, 0] = start_rs(q, 0)
        gemm1_chunk((my + 2) % N_DEV)
        gemm1_chunk(my)

        ag = {}
        for s in range(N_DEV - 1):
            for q in CHAINS:
                rs[q, s].wait()
                c = rs_recv_idx(q, s)
                h_ref[pl.ds(row0(c, q), q_rows), :] = (
                    rs_recv[q, s] + h_ref[pl.ds(row0(c, q), q_rows), :]
                )
                if s < N_DEV - 2:
                    rs[q, s + 1] = start_rs(q, s + 1)
                else:
                    ag[q, 0] = start_ag(q, 0)

        for q in CHAINS:
            gemm2_store(row0(my, q), h_ref[pl.ds(row0(my, q), q_rows), :])

        for t in range(N_DEV - 1):
            for q in CHAINS:
                ag[q, t].wait()
                if t < N_DEV - 2:
                    ag[q, t + 1] = start_ag(q, t + 1)
                gemm2_store(row0(ag_hold_idx(q, t + 1), q), ag_recv[q, t])

    return pl.pallas_call(
        body,
        out_shape=jax.ShapeDtypeStruct((m, n), jnp.bfloat16),
        in_specs=[
            pl.BlockSpec(memory_space=pltpu.VMEM),
            pl.BlockSpec(memory_space=pltpu.VMEM),
            pl.BlockSpec(memory_space=pltpu.VMEM),
        ],
        out_specs=pl.BlockSpec(memory_space=pltpu.VMEM),
        scratch_shapes=[
            pltpu.VMEM((m, d), jnp.bfloat16),
            pltpu.VMEM((N_Q, N_DEV - 1, q_rows, d), jnp.bfloat16),
            pltpu.VMEM((N_Q, N_DEV - 1, q_rows, d), jnp.bfloat16),
            pltpu.SemaphoreType.DMA((N_Q, N_DEV - 1)),
            pltpu.SemaphoreType.DMA((N_Q, N_DEV - 1)),
            pltpu.SemaphoreType.DMA((N_Q, N_DEV - 1)),
            pltpu.SemaphoreType.DMA((N_Q, N_DEV - 1)),
        ],
        compiler_params=_CompilerParams(
            collective_id=0, vmem_limit_bytes=42 * 1024 * 1024
        ),
    )(xb, w1b, w2b)
